# baseline (speedup 1.0000x reference)
import functools

import jax
import jax.numpy as jnp
from jax import lax
from jax.experimental import pallas as pl
from jax.experimental.pallas import tpu as pltpu
from jax.experimental.pallas import tpu_sc as plsc

N_NODES = 100000
N_EDGES = 1600000
B = 10000
NBR = 20
NODE_DIM = 128
EDGE_DIM = 16
TIME_DIM = 16
QDIM = NODE_DIM + TIME_DIM
KDIM = NODE_DIM + TIME_DIM + EDGE_DIM
N_HEADS = 2
DH = QDIM // N_HEADS

NW = 32
NODE_TOT = 215040
NODE_PW = NODE_TOT // NW
NODE_CH = 280
NODE_NCH = NODE_PW // NODE_CH
EDGE_TOT = 204800
EDGE_PW = EDGE_TOT // NW
EDGE_CH = 800
EDGE_NCH = EDGE_PW // EDGE_CH

RB = 40
NRB = RB * NBR
NBLK = B // RB


def _combine_body(m_ref, f_ref, o_ref):
    o_ref[...] = m_ref[...] + f_ref[...]


def _sc_gather_body(comb_hbm, ef_hbm, nidx_hbm, eidx_hbm,
                    nrows_out, erows_out,
                    nidx_v, eidx_v, nbuf, ebuf, nsem, esem):
    wid = lax.axis_index("s") * 2 + lax.axis_index("c")
    nbase = wid * NODE_PW
    ebase = wid * EDGE_PW
    pltpu.sync_copy(nidx_hbm.at[pl.ds(nbase, NODE_PW)], nidx_v)
    pltpu.sync_copy(eidx_hbm.at[pl.ds(ebase, EDGE_PW)], eidx_v)

    def node_step(i, _):
        off = i * NODE_CH
        pltpu.async_copy(
            comb_hbm.at[nidx_v.at[pl.ds(off, NODE_CH)]], nbuf, nsem).wait()
        pltpu.sync_copy(nbuf, nrows_out.at[pl.ds(nbase + off, NODE_CH)])
        return 0

    def edge_step(i, _):
        off = i * EDGE_CH
        pltpu.async_copy(
            ef_hbm.at[eidx_v.at[pl.ds(off, EDGE_CH)]], ebuf, esem).wait()
        pltpu.sync_copy(ebuf, erows_out.at[pl.ds(ebase + off, EDGE_CH)])
        return 0

    lax.fori_loop(0, NODE_NCH, node_step, 0)
    lax.fori_loop(0, EDGE_NCH, edge_step, 0)


def _attn_body(nbr_ref, src_ref, ef_ref, et_ref, ts_ref, nb_ref,
               tw_ref, tb_ref,
               wq1_ref, wq2_ref, bq_ref,
               wk1_ref, wk2_ref, wk3_ref, bk_ref,
               wv1_ref, wv2_ref, wv3_ref, bv_ref,
               wo_ref, bo_ref,
               fc1a_ref, fc1b_ref, fc1b_b_ref, fc2_ref, fc2b_ref,
               out_ref):
    f32 = jnp.float32
    rows = lax.broadcasted_iota(jnp.int32, (NRB, RB), 0) // NBR
    cols = lax.broadcasted_iota(jnp.int32, (NRB, RB), 1)
    bd = (rows == cols).astype(f32)
    rows_t = lax.broadcasted_iota(jnp.int32, (RB, NRB), 1) // NBR
    cols_t = lax.broadcasted_iota(jnp.int32, (RB, NRB), 0)
    bdt = (rows_t == cols_t).astype(f32)
    dsel = lax.broadcasted_iota(jnp.int32, (QDIM, N_HEADS), 0) // DH
    hsel = lax.broadcasted_iota(jnp.int32, (QDIM, N_HEADS), 1)
    seg = (dsel == hsel).astype(f32)
    dsel2 = lax.broadcasted_iota(jnp.int32, (N_HEADS, QDIM), 1) // DH
    hsel2 = lax.broadcasted_iota(jnp.int32, (N_HEADS, QDIM), 0)
    seg_t = (dsel2 == hsel2).astype(f32)

    dot = functools.partial(jnp.dot, preferred_element_type=f32,
                            precision=lax.Precision.HIGHEST)

    nbr = nbr_ref[...]
    src = src_ref[...]
    ef = ef_ref[...]
    et = et_ref[...]
    ts = ts_ref[...]
    nb = nb_ref[...]

    deltas = dot(bd, ts) - et
    et_enc = jnp.cos(deltas * tw_ref[...] + tb_ref[...])
    st_row = jnp.cos(tb_ref[...])

    q = dot(src, wq1_ref[...]) + dot(st_row, wq2_ref[...]) + bq_ref[...]
    k = (dot(nbr, wk1_ref[...]) + dot(et_enc, wk2_ref[...])
         + dot(ef, wk3_ref[...]) + bk_ref[...])
    v = (dot(nbr, wv1_ref[...]) + dot(et_enc, wv2_ref[...])
         + dot(ef, wv3_ref[...]) + bv_ref[...])

    q_rep = dot(bd, q)
    scores = dot(q_rep * k, seg) * (1.0 / (DH ** 0.5))
    masked = (nb == 0)
    scores = jnp.where(masked, -1e9, scores)
    e = jnp.exp(scores)
    den = dot(bdt, e)
    den = jnp.where(den == 0.0, 1.0, den)
    attn = e * dot(bd, 1.0 / den)
    av = dot(attn, seg_t) * v
    outh = dot(bdt, av)
    out = dot(outh, wo_ref[...]) + bo_ref[...]
    valid = jnp.where(masked, 0.0, 1.0)
    nvalid = dot(bdt, valid)
    out = jnp.where(nvalid == 0.0, 0.0, out)

    h1 = jnp.maximum(
        dot(out, fc1a_ref[...]) + dot(src, fc1b_ref[...]) + fc1b_b_ref[...],
        0.0)
    out_ref[...] = dot(h1, fc2_ref[...]) + fc2b_ref[...]


def kernel(memory, node_features, edge_features, timestamps, edge_times,
           time_w, time_b, Wq, bq, Wk, bk, Wv, bv, Wo, bo,
           fc1_w, fc1_b, fc2_w, fc2_b, src_nodes, neighbors, edge_idxs):
    f32 = jnp.float32

    combined = pl.pallas_call(
        _combine_body,
        out_shape=jax.ShapeDtypeStruct((N_NODES, NODE_DIM), f32),
        grid=(50,),
        in_specs=[pl.BlockSpec((2000, NODE_DIM), lambda i: (i, 0)),
                  pl.BlockSpec((2000, NODE_DIM), lambda i: (i, 0))],
        out_specs=pl.BlockSpec((2000, NODE_DIM), lambda i: (i, 0)),
    )(memory, node_features)

    flat_nbr = neighbors.reshape(-1).astype(jnp.int32)
    node_idx = jnp.concatenate([
        flat_nbr, src_nodes.astype(jnp.int32),
        jnp.zeros((NODE_TOT - B * NBR - B,), jnp.int32)])
    edge_idx = jnp.concatenate([
        edge_idxs.reshape(-1).astype(jnp.int32),
        jnp.zeros((EDGE_TOT - B * NBR,), jnp.int32)])

    mesh = plsc.VectorSubcoreMesh(core_axis_name="c", subcore_axis_name="s")
    node_rows, edge_rows = pl.kernel(
        _sc_gather_body,
        out_type=[jax.ShapeDtypeStruct((NODE_TOT, NODE_DIM), f32),
                  jax.ShapeDtypeStruct((EDGE_TOT, EDGE_DIM), f32)],
        mesh=mesh,
        compiler_params=pltpu.CompilerParams(use_tc_tiling_on_sc=False),
        scratch_types=[
            pltpu.VMEM((NODE_PW,), jnp.int32),
            pltpu.VMEM((EDGE_PW,), jnp.int32),
            pltpu.VMEM((NODE_CH, NODE_DIM), f32),
            pltpu.VMEM((EDGE_CH, EDGE_DIM), f32),
            pltpu.SemaphoreType.DMA,
            pltpu.SemaphoreType.DMA,
        ],
    )(combined, edge_features, node_idx, edge_idx)

    et_flat = edge_times.reshape(B * NBR, 1).astype(f32)
    ts_col = timestamps.reshape(B, 1).astype(f32)
    nb_flat = neighbors.reshape(B * NBR, 1).astype(jnp.int32)

    tw = time_w.reshape(1, TIME_DIM)
    tb = time_b.reshape(1, TIME_DIM)
    wq1 = Wq[:, :NODE_DIM].T
    wq2 = Wq[:, NODE_DIM:].T
    wk1 = Wk[:, :NODE_DIM].T
    wk2 = Wk[:, NODE_DIM:NODE_DIM + TIME_DIM].T
    wk3 = Wk[:, NODE_DIM + TIME_DIM:].T
    wv1 = Wv[:, :NODE_DIM].T
    wv2 = Wv[:, NODE_DIM:NODE_DIM + TIME_DIM].T
    wv3 = Wv[:, NODE_DIM + TIME_DIM:].T
    wo_t = Wo.T
    fc1a = fc1_w[:, :QDIM].T
    fc1b = fc1_w[:, QDIM:].T
    fc2t = fc2_w.T

    def full(a):
        a2 = a.reshape((1, -1)) if a.ndim == 1 else a
        return a2, pl.BlockSpec(a2.shape, lambda i: tuple(0 for _ in a2.shape))

    const_args = [tw, tb, wq1, wq2, bq, wk1, wk2, wk3, bk,
                  wv1, wv2, wv3, bv, wo_t, bo, fc1a, fc1b, fc1_b, fc2t, fc2_b]
    const_vals, const_specs = zip(*[full(a) for a in const_args])

    out = pl.pallas_call(
        _attn_body,
        out_shape=jax.ShapeDtypeStruct((B, NODE_DIM), f32),
        grid=(NBLK,),
        in_specs=[
            pl.BlockSpec((NRB, NODE_DIM), lambda i: (i, 0)),
            pl.BlockSpec((RB, NODE_DIM), lambda i: (B * NBR // RB + i, 0)),
            pl.BlockSpec((NRB, EDGE_DIM), lambda i: (i, 0)),
            pl.BlockSpec((NRB, 1), lambda i: (i, 0)),
            pl.BlockSpec((RB, 1), lambda i: (i, 0)),
            pl.BlockSpec((NRB, 1), lambda i: (i, 0)),
        ] + list(const_specs),
        out_specs=pl.BlockSpec((RB, NODE_DIM), lambda i: (i, 0)),
    )(node_rows, node_rows, edge_rows, et_flat, ts_col, nb_flat, *const_vals)
    return out


# speedup vs baseline: 1.1866x; 1.1866x over previous
import functools

import jax
import jax.numpy as jnp
from jax import lax
from jax.experimental import pallas as pl
from jax.experimental.pallas import tpu as pltpu
from jax.experimental.pallas import tpu_sc as plsc

N_NODES = 100000
N_EDGES = 1600000
B = 10000
NBR = 20
NODE_DIM = 128
EDGE_DIM = 16
TIME_DIM = 16
QDIM = NODE_DIM + TIME_DIM
KDIM = NODE_DIM + TIME_DIM + EDGE_DIM
N_HEADS = 2
DH = QDIM // N_HEADS

NW = 32
NODE_TOT = 215040
NODE_PW = NODE_TOT // NW
NODE_CH = 280
NODE_NCH = NODE_PW // NODE_CH
EDGE_TOT = 204800
EDGE_PW = EDGE_TOT // NW
EDGE_CH = 800
EDGE_NCH = EDGE_PW // EDGE_CH

RB = 40
NRB = RB * NBR
NBLK = B // RB


def _combine_body(m_ref, f_ref, o_ref):
    o_ref[...] = m_ref[...] + f_ref[...]


def _sc_node_body(comb_hbm, nidx_hbm, nrows_out, nidx_v, nbuf0, nbuf1,
                  sem0, sem1):
    wid = lax.axis_index("s") * 2 + lax.axis_index("c")
    nbase = wid * NODE_PW
    pltpu.sync_copy(nidx_hbm.at[pl.ds(nbase, NODE_PW)], nidx_v)
    bufs = (nbuf0, nbuf1)
    sems = (sem0, sem1)
    pltpu.async_copy(comb_hbm.at[nidx_v.at[pl.ds(0, NODE_CH)]], nbuf0, sem0)
    pltpu.async_copy(
        comb_hbm.at[nidx_v.at[pl.ds(NODE_CH, NODE_CH)]], nbuf1, sem1)

    @pl.loop(0, NODE_NCH, step=2)
    def _(g):
        for p in range(2):
            c = g + p
            buf, sem = bufs[p], sems[p]
            pltpu.make_async_copy(
                comb_hbm.at[pl.ds(0, NODE_CH)], buf, sem).wait()
            pltpu.sync_copy(
                buf, nrows_out.at[pl.ds(nbase + c * NODE_CH, NODE_CH)])

            @pl.when(c + 2 < NODE_NCH)
            def _issue():
                off = (c + 2) * NODE_CH
                pltpu.async_copy(
                    comb_hbm.at[nidx_v.at[pl.ds(off, NODE_CH)]], buf, sem)


def _sc_edge_body(ef_hbm, eidx_hbm, erows_out, eidx_v, ebuf0, ebuf1,
                  sem0, sem1):
    wid = lax.axis_index("s") * 2 + lax.axis_index("c")
    ebase = wid * EDGE_PW
    pltpu.sync_copy(eidx_hbm.at[pl.ds(ebase, EDGE_PW)], eidx_v)
    bufs = (ebuf0, ebuf1)
    sems = (sem0, sem1)
    for c in range(2):
        pltpu.async_copy(
            ef_hbm.at[eidx_v.at[pl.ds(c * EDGE_CH, EDGE_CH)]], bufs[c],
            sems[c])
    for c in range(EDGE_NCH):
        buf, sem = bufs[c % 2], sems[c % 2]
        pltpu.make_async_copy(ef_hbm.at[pl.ds(0, EDGE_CH)], buf, sem).wait()
        pltpu.sync_copy(buf, erows_out.at[pl.ds(ebase + c * EDGE_CH, EDGE_CH)])
        if c + 2 < EDGE_NCH:
            off = (c + 2) * EDGE_CH
            pltpu.async_copy(
                ef_hbm.at[eidx_v.at[pl.ds(off, EDGE_CH)]], buf, sem)


def _attn_body(nbr_ref, src_ref, ef_ref, et_ref, ts_ref, nb_ref,
               tw_ref, tb_ref,
               wq1_ref, wq2_ref, bq_ref,
               wk_ref, bk_ref, wv_ref, bv_ref,
               wo_ref, bo_ref,
               fc1a_ref, fc1b_ref, fc1b_b_ref, fc2_ref, fc2b_ref,
               out_ref):
    f32 = jnp.float32
    rows = lax.broadcasted_iota(jnp.int32, (NRB, RB), 0) // NBR
    cols = lax.broadcasted_iota(jnp.int32, (NRB, RB), 1)
    bd = (rows == cols).astype(f32)
    rows_t = lax.broadcasted_iota(jnp.int32, (RB, NRB), 1) // NBR
    cols_t = lax.broadcasted_iota(jnp.int32, (RB, NRB), 0)
    bdt = (rows_t == cols_t).astype(f32)
    dsel = lax.broadcasted_iota(jnp.int32, (QDIM, N_HEADS), 0) // DH
    hsel = lax.broadcasted_iota(jnp.int32, (QDIM, N_HEADS), 1)
    seg = (dsel == hsel).astype(f32)
    dsel2 = lax.broadcasted_iota(jnp.int32, (N_HEADS, QDIM), 1) // DH
    hsel2 = lax.broadcasted_iota(jnp.int32, (N_HEADS, QDIM), 0)
    seg_t = (dsel2 == hsel2).astype(f32)

    dot = functools.partial(jnp.dot, preferred_element_type=f32,
                            precision=lax.Precision.HIGHEST)

    nbr = nbr_ref[...]
    src = src_ref[...]
    ef = ef_ref[...]
    et = et_ref[...]
    ts = ts_ref[...]
    nb = nb_ref[...]

    deltas = dot(bd, ts) - et
    et_enc = jnp.cos(deltas * tw_ref[...] + tb_ref[...])
    st_row = jnp.cos(tb_ref[...])

    q = dot(src, wq1_ref[...]) + dot(st_row, wq2_ref[...]) + bq_ref[...]
    keyk = jnp.concatenate([nbr, et_enc, ef], axis=1)
    k = dot(keyk, wk_ref[...]) + bk_ref[...]
    v = dot(keyk, wv_ref[...]) + bv_ref[...]

    q_rep = dot(bd, q)
    scores = dot(q_rep * k, seg) * (1.0 / (DH ** 0.5))
    masked = (nb == 0)
    scores = jnp.where(masked, -1e9, scores)
    e = jnp.exp(scores)
    den = dot(bdt, e)
    den = jnp.where(den == 0.0, 1.0, den)
    attn = e * dot(bd, 1.0 / den)
    av = dot(attn, seg_t) * v
    outh = dot(bdt, av)
    out = dot(outh, wo_ref[...]) + bo_ref[...]
    valid = jnp.where(masked, 0.0, 1.0)
    nvalid = dot(bdt, valid)
    out = jnp.where(nvalid == 0.0, 0.0, out)

    h1 = jnp.maximum(
        dot(out, fc1a_ref[...]) + dot(src, fc1b_ref[...]) + fc1b_b_ref[...],
        0.0)
    out_ref[...] = dot(h1, fc2_ref[...]) + fc2b_ref[...]


def kernel(memory, node_features, edge_features, timestamps, edge_times,
           time_w, time_b, Wq, bq, Wk, bk, Wv, bv, Wo, bo,
           fc1_w, fc1_b, fc2_w, fc2_b, src_nodes, neighbors, edge_idxs):
    f32 = jnp.float32

    combined = pl.pallas_call(
        _combine_body,
        out_shape=jax.ShapeDtypeStruct((N_NODES, NODE_DIM), f32),
        grid=(50,),
        in_specs=[pl.BlockSpec((2000, NODE_DIM), lambda i: (i, 0)),
                  pl.BlockSpec((2000, NODE_DIM), lambda i: (i, 0))],
        out_specs=pl.BlockSpec((2000, NODE_DIM), lambda i: (i, 0)),
    )(memory, node_features)

    flat_nbr = neighbors.reshape(-1).astype(jnp.int32)
    node_idx = jnp.concatenate([
        flat_nbr, src_nodes.astype(jnp.int32),
        jnp.zeros((NODE_TOT - B * NBR - B,), jnp.int32)])
    edge_idx = jnp.concatenate([
        edge_idxs.reshape(-1).astype(jnp.int32),
        jnp.zeros((EDGE_TOT - B * NBR,), jnp.int32)])

    mesh = plsc.VectorSubcoreMesh(core_axis_name="c", subcore_axis_name="s")
    node_rows = pl.kernel(
        _sc_node_body,
        out_type=jax.ShapeDtypeStruct((NODE_TOT, NODE_DIM), f32),
        mesh=mesh,
        scratch_types=[
            pltpu.VMEM((NODE_PW,), jnp.int32),
            pltpu.VMEM((NODE_CH, NODE_DIM), f32),
            pltpu.VMEM((NODE_CH, NODE_DIM), f32),
            pltpu.SemaphoreType.DMA,
            pltpu.SemaphoreType.DMA,
        ],
    )(combined, node_idx)

    edge_rows = pl.kernel(
        _sc_edge_body,
        out_type=jax.ShapeDtypeStruct((EDGE_TOT, EDGE_DIM), f32),
        mesh=mesh,
        compiler_params=pltpu.CompilerParams(use_tc_tiling_on_sc=False),
        scratch_types=[
            pltpu.VMEM((EDGE_PW,), jnp.int32),
            pltpu.VMEM((EDGE_CH, EDGE_DIM), f32),
            pltpu.VMEM((EDGE_CH, EDGE_DIM), f32),
            pltpu.SemaphoreType.DMA,
            pltpu.SemaphoreType.DMA,
        ],
    )(edge_features, edge_idx)

    et_flat = edge_times.reshape(B * NBR, 1).astype(f32)
    ts_col = timestamps.reshape(B, 1).astype(f32)
    nb_flat = neighbors.reshape(B * NBR, 1).astype(jnp.int32)

    tw = time_w.reshape(1, TIME_DIM)
    tb = time_b.reshape(1, TIME_DIM)
    wq1 = Wq[:, :NODE_DIM].T
    wq2 = Wq[:, NODE_DIM:].T
    wk_t = Wk.T
    wv_t = Wv.T
    wo_t = Wo.T
    fc1a = fc1_w[:, :QDIM].T
    fc1b = fc1_w[:, QDIM:].T
    fc2t = fc2_w.T

    def full(a):
        a2 = a.reshape((1, -1)) if a.ndim == 1 else a
        return a2, pl.BlockSpec(a2.shape, lambda i: tuple(0 for _ in a2.shape))

    const_args = [tw, tb, wq1, wq2, bq, wk_t, bk, wv_t, bv,
                  wo_t, bo, fc1a, fc1b, fc1_b, fc2t, fc2_b]
    const_vals, const_specs = zip(*[full(a) for a in const_args])

    out = pl.pallas_call(
        _attn_body,
        out_shape=jax.ShapeDtypeStruct((B, NODE_DIM), f32),
        grid=(NBLK,),
        in_specs=[
            pl.BlockSpec((NRB, NODE_DIM), lambda i: (i, 0)),
            pl.BlockSpec((RB, NODE_DIM), lambda i: (B * NBR // RB + i, 0)),
            pl.BlockSpec((NRB, EDGE_DIM), lambda i: (i, 0)),
            pl.BlockSpec((NRB, 1), lambda i: (i, 0)),
            pl.BlockSpec((RB, 1), lambda i: (i, 0)),
            pl.BlockSpec((NRB, 1), lambda i: (i, 0)),
        ] + list(const_specs),
        out_specs=pl.BlockSpec((RB, NODE_DIM), lambda i: (i, 0)),
    )(node_rows, node_rows, edge_rows, et_flat, ts_col, nb_flat, *const_vals)
    return out


# speedup vs baseline: 1.7597x; 1.4830x over previous
import functools

import jax
import jax.numpy as jnp
from jax import lax
from jax.experimental import pallas as pl
from jax.experimental.pallas import tpu as pltpu
from jax.experimental.pallas import tpu_sc as plsc

N_NODES = 100000
N_EDGES = 1600000
B = 10000
NBR = 20
NODE_DIM = 128
EDGE_DIM = 16
TIME_DIM = 16
QDIM = NODE_DIM + TIME_DIM
N_HEADS = 2
DH = QDIM // N_HEADS
EPR = NODE_DIM // EDGE_DIM

NW = 32
NODE_TOT = 215040
NODE_PW = NODE_TOT // NW
NODE_CH = 240
NODE_NCH = NODE_PW // NODE_CH
EW_TOT = 204800
EW_PW = EW_TOT // NW
EW_CH = 160
EW_NCH = EW_PW // EW_CH

RB = 80
NRB = RB * NBR
NBLK = B // RB


def _combine_body(m_ref, f_ref, o_ref):
    o_ref[...] = m_ref[...] + f_ref[...]


def _sc_gather_body(comb_hbm, efw_hbm, nidx_hbm, eidx_hbm,
                    nrows_out, erows_out,
                    nidx_v, eidx_v, nbuf0, nbuf1, ebuf0, ebuf1,
                    nsem0, nsem1, esem0, esem1):
    wid = lax.axis_index("s") * 2 + lax.axis_index("c")
    nbase = wid * NODE_PW
    ebase = wid * EW_PW
    pltpu.sync_copy(nidx_hbm.at[pl.ds(nbase, NODE_PW)], nidx_v)
    pltpu.sync_copy(eidx_hbm.at[pl.ds(ebase, EW_PW)], eidx_v)
    nbufs, nsems = (nbuf0, nbuf1), (nsem0, nsem1)
    ebufs, esems = (ebuf0, ebuf1), (esem0, esem1)
    for p in range(2):
        pltpu.async_copy(
            comb_hbm.at[nidx_v.at[pl.ds(p * NODE_CH, NODE_CH)]],
            nbufs[p], nsems[p])
        pltpu.async_copy(
            efw_hbm.at[eidx_v.at[pl.ds(p * EW_CH, EW_CH)]],
            ebufs[p], esems[p])

    @pl.loop(0, EW_NCH, step=2)
    def _(g):
        for p in range(2):
            c = g + p

            @pl.when(c < NODE_NCH)
            def _node():
                buf, sem = nbufs[p], nsems[p]
                pltpu.make_async_copy(
                    comb_hbm.at[pl.ds(0, NODE_CH)], buf, sem).wait()
                pltpu.sync_copy(
                    buf, nrows_out.at[pl.ds(nbase + c * NODE_CH, NODE_CH)])

                @pl.when(c + 2 < NODE_NCH)
                def _issue_n():
                    off = (c + 2) * NODE_CH
                    pltpu.async_copy(
                        comb_hbm.at[nidx_v.at[pl.ds(off, NODE_CH)]], buf, sem)

            buf, sem = ebufs[p], esems[p]
            pltpu.make_async_copy(
                efw_hbm.at[pl.ds(0, EW_CH)], buf, sem).wait()
            pltpu.sync_copy(
                buf, erows_out.at[pl.ds(ebase + c * EW_CH, EW_CH)])

            @pl.when(c + 2 < EW_NCH)
            def _issue_e():
                off = (c + 2) * EW_CH
                pltpu.async_copy(
                    efw_hbm.at[eidx_v.at[pl.ds(off, EW_CH)]], buf, sem)


def _attn_body(nbr_ref, src_ref, efw_ref, et_ref, ts_ref, nb_ref, eix_ref,
               tw_ref, tb_ref,
               wq1_ref, wq2_ref, bq_ref,
               wk_ref, bk_ref, wv_ref, bv_ref,
               wo_ref, bo_ref,
               fc1a_ref, fc1b_ref, fc1b_b_ref, fc2_ref, fc2b_ref,
               out_ref):
    f32 = jnp.float32
    i32 = jnp.int32
    rows = lax.broadcasted_iota(i32, (NRB, RB), 0) // NBR
    cols = lax.broadcasted_iota(i32, (NRB, RB), 1)
    bd = (rows == cols).astype(f32)
    rows_t = lax.broadcasted_iota(i32, (RB, NRB), 1) // NBR
    cols_t = lax.broadcasted_iota(i32, (RB, NRB), 0)
    bdt = (rows_t == cols_t).astype(f32)
    dsel = lax.broadcasted_iota(i32, (QDIM, N_HEADS), 0) // DH
    hsel = lax.broadcasted_iota(i32, (QDIM, N_HEADS), 1)
    seg = (dsel == hsel).astype(f32)
    dsel2 = lax.broadcasted_iota(i32, (N_HEADS, QDIM), 1) // DH
    hsel2 = lax.broadcasted_iota(i32, (N_HEADS, QDIM), 0)
    seg_t = (dsel2 == hsel2).astype(f32)

    dotH = functools.partial(jnp.dot, preferred_element_type=f32,
                             precision=lax.Precision.HIGHEST)
    dotD = functools.partial(jnp.dot, preferred_element_type=f32)

    nbr = nbr_ref[...]
    src = src_ref[...]
    efw = efw_ref[...]
    et = et_ref[...]
    ts = ts_ref[...]
    nb = nb_ref[...]
    eix = eix_ref[...]

    lane = lax.broadcasted_iota(i32, (NRB, NODE_DIM), 1)
    ef_sel = jnp.where((lane // EDGE_DIM) == (eix & (EPR - 1)), efw, 0.0)
    psel = (lax.broadcasted_iota(i32, (NODE_DIM, EDGE_DIM), 0) % EDGE_DIM
            == lax.broadcasted_iota(i32, (NODE_DIM, EDGE_DIM), 1))
    ef = dotD(ef_sel, psel.astype(f32))

    deltas = dotH(bd, ts) - et
    et_enc = jnp.cos(deltas * tw_ref[...] + tb_ref[...])
    st_row = jnp.cos(tb_ref[...])

    q = dotH(src, wq1_ref[...]) + dotH(st_row, wq2_ref[...]) + bq_ref[...]
    keyk = jnp.concatenate([nbr, et_enc, ef], axis=1)
    k = dotH(keyk, wk_ref[...]) + bk_ref[...]
    v = dotH(keyk, wv_ref[...]) + bv_ref[...]

    q_rep = dotD(bd, q)
    scores = dotD(q_rep * k, seg) * (1.0 / (DH ** 0.5))
    masked = (nb == 0)
    scores = jnp.where(masked, -1e9, scores)
    e = jnp.exp(scores)
    den = dotD(bdt, e)
    fully_masked = den[:, 0:1] == 0.0
    den = jnp.where(den == 0.0, 1.0, den)
    attn = e * dotD(bd, 1.0 / den)
    av = dotD(attn, seg_t) * v
    outh = dotD(bdt, av)
    out = dotH(outh, wo_ref[...]) + bo_ref[...]
    out = jnp.where(fully_masked, 0.0, out)

    h1 = jnp.maximum(
        dotH(out, fc1a_ref[...]) + dotH(src, fc1b_ref[...]) + fc1b_b_ref[...],
        0.0)
    out_ref[...] = dotH(h1, fc2_ref[...]) + fc2b_ref[...]


def kernel(memory, node_features, edge_features, timestamps, edge_times,
           time_w, time_b, Wq, bq, Wk, bk, Wv, bv, Wo, bo,
           fc1_w, fc1_b, fc2_w, fc2_b, src_nodes, neighbors, edge_idxs):
    f32 = jnp.float32

    combined = pl.pallas_call(
        _combine_body,
        out_shape=jax.ShapeDtypeStruct((N_NODES, NODE_DIM), f32),
        grid=(50,),
        in_specs=[pl.BlockSpec((2000, NODE_DIM), lambda i: (i, 0)),
                  pl.BlockSpec((2000, NODE_DIM), lambda i: (i, 0))],
        out_specs=pl.BlockSpec((2000, NODE_DIM), lambda i: (i, 0)),
    )(memory, node_features)

    flat_nbr = neighbors.reshape(-1).astype(jnp.int32)
    flat_eix = edge_idxs.reshape(-1).astype(jnp.int32)
    node_idx = jnp.concatenate([
        flat_nbr, src_nodes.astype(jnp.int32),
        jnp.zeros((NODE_TOT - B * NBR - B,), jnp.int32)])
    ew_idx = jnp.concatenate([
        flat_eix // EPR, jnp.zeros((EW_TOT - B * NBR,), jnp.int32)])
    efw = edge_features.reshape(N_EDGES // EPR, NODE_DIM)

    mesh = plsc.VectorSubcoreMesh(core_axis_name="c", subcore_axis_name="s")
    node_rows, ew_rows = pl.kernel(
        _sc_gather_body,
        out_type=[jax.ShapeDtypeStruct((NODE_TOT, NODE_DIM), f32),
                  jax.ShapeDtypeStruct((EW_TOT, NODE_DIM), f32)],
        mesh=mesh,
        scratch_types=[
            pltpu.VMEM((NODE_PW,), jnp.int32),
            pltpu.VMEM((EW_PW,), jnp.int32),
            pltpu.VMEM((NODE_CH, NODE_DIM), f32),
            pltpu.VMEM((NODE_CH, NODE_DIM), f32),
            pltpu.VMEM((EW_CH, NODE_DIM), f32),
            pltpu.VMEM((EW_CH, NODE_DIM), f32),
            pltpu.SemaphoreType.DMA,
            pltpu.SemaphoreType.DMA,
            pltpu.SemaphoreType.DMA,
            pltpu.SemaphoreType.DMA,
        ],
    )(combined, efw, node_idx, ew_idx)

    et_flat = edge_times.reshape(B * NBR, 1).astype(f32)
    ts_col = timestamps.reshape(B, 1).astype(f32)
    nb_flat = neighbors.reshape(B * NBR, 1).astype(jnp.int32)
    eix_flat = flat_eix.reshape(B * NBR, 1)

    tw = time_w.reshape(1, TIME_DIM)
    tb = time_b.reshape(1, TIME_DIM)
    wq1 = Wq[:, :NODE_DIM].T
    wq2 = Wq[:, NODE_DIM:].T
    wk_t = Wk.T
    wv_t = Wv.T
    wo_t = Wo.T
    fc1a = fc1_w[:, :QDIM].T
    fc1b = fc1_w[:, QDIM:].T
    fc2t = fc2_w.T

    def full(a):
        a2 = a.reshape((1, -1)) if a.ndim == 1 else a
        return a2, pl.BlockSpec(a2.shape, lambda i: tuple(0 for _ in a2.shape))

    const_args = [tw, tb, wq1, wq2, bq, wk_t, bk, wv_t, bv,
                  wo_t, bo, fc1a, fc1b, fc1_b, fc2t, fc2_b]
    const_vals, const_specs = zip(*[full(a) for a in const_args])

    out = pl.pallas_call(
        _attn_body,
        out_shape=jax.ShapeDtypeStruct((B, NODE_DIM), f32),
        grid=(NBLK,),
        in_specs=[
            pl.BlockSpec((NRB, NODE_DIM), lambda i: (i, 0)),
            pl.BlockSpec((RB, NODE_DIM), lambda i: (B * NBR // RB + i, 0)),
            pl.BlockSpec((NRB, NODE_DIM), lambda i: (i, 0)),
            pl.BlockSpec((NRB, 1), lambda i: (i, 0)),
            pl.BlockSpec((RB, 1), lambda i: (i, 0)),
            pl.BlockSpec((NRB, 1), lambda i: (i, 0)),
            pl.BlockSpec((NRB, 1), lambda i: (i, 0)),
        ] + list(const_specs),
        out_specs=pl.BlockSpec((RB, NODE_DIM), lambda i: (i, 0)),
    )(node_rows, node_rows, ew_rows, et_flat, ts_col, nb_flat, eix_flat,
      *const_vals)
    return out
